# Initial kernel scaffold; baseline (speedup 1.0000x reference)
import jax
import jax.numpy as jnp
from jax.experimental import pallas as pl


def kernel(features_nir, features_raman, W_nir, b_nir, W_raman, b_raman, pos_table, spec_table):
    raise NotImplementedError("write your pallas kernel here")


# speedup vs baseline: 10.8038x; 10.8038x over previous
import jax
import jax.numpy as jnp
from jax.experimental import pallas as pl


def _tok_kernel(fn_ref, fr_ref, wn_ref, wr_ref, cn_ref, cr_ref, pos_ref, out_ref):
    n = pos_ref.shape[0]
    base_n = pos_ref[...] + cn_ref[0, :][None, :]
    base_r = pos_ref[...] + cr_ref[0, :][None, :]
    w_n = wn_ref[0, :]
    w_r = wr_ref[0, :]
    out_ref[:, :n, :] = fn_ref[...][:, :, None] * w_n[None, None, :] + base_n[None, :, :]
    out_ref[:, n:, :] = fr_ref[...][:, :, None] * w_r[None, None, :] + base_r[None, :, :]


def kernel(features_nir, features_raman, W_nir, b_nir, W_raman, b_raman, pos_table, spec_table):
    B, N = features_nir.shape
    D = pos_table.shape[1]
    TB = 32

    w_n = W_nir[:, 0][None, :]
    w_r = W_raman[:, 0][None, :]
    const_n = (b_nir + spec_table[0])[None, :]
    const_r = (b_raman + spec_table[1])[None, :]

    grid = (B // TB,)
    out = pl.pallas_call(
        _tok_kernel,
        grid=grid,
        in_specs=[
            pl.BlockSpec((TB, N), lambda i: (i, 0)),
            pl.BlockSpec((TB, N), lambda i: (i, 0)),
            pl.BlockSpec((1, D), lambda i: (0, 0)),
            pl.BlockSpec((1, D), lambda i: (0, 0)),
            pl.BlockSpec((1, D), lambda i: (0, 0)),
            pl.BlockSpec((1, D), lambda i: (0, 0)),
            pl.BlockSpec((N, D), lambda i: (0, 0)),
        ],
        out_specs=pl.BlockSpec((TB, 2 * N, D), lambda i: (i, 0, 0)),
        out_shape=jax.ShapeDtypeStruct((B, 2 * N, D), features_nir.dtype),
    )(features_nir, features_raman, w_n, w_r, const_n, const_r, pos_table)
    return out
